# baseline (speedup 1.0000x reference)
import functools

import jax
import jax.numpy as jnp
from jax import lax
from jax.experimental import pallas as pl
from jax.experimental.pallas import tpu as pltpu
from jax.experimental.pallas import tpu_sc as plsc

S = 7
NCELL = S * S
D = 30
BATCH = 64
NPAD = 64
DET_PAD = 320
CONF_THRES = 0.5
NMS_THRES = 0.7
GRID = 64.0
WIMG = 448.0
NEG_INF = float("-inf")

_mesh = plsc.VectorSubcoreMesh(core_axis_name="c", subcore_axis_name="s")


@functools.partial(
    pl.kernel,
    out_type=(
        jax.ShapeDtypeStruct((BATCH, DET_PAD), jnp.float32),
        jax.ShapeDtypeStruct((BATCH, 2 * NPAD), jnp.int32),
    ),
    mesh=_mesh,
    compiler_params=pltpu.CompilerParams(needs_layout_passes=False),
    scratch_types=[
        pltpu.VMEM((2, D * NPAD), jnp.float32),
        pltpu.VMEM((128,), jnp.float32),
        pltpu.VMEM((128,), jnp.float32),
        pltpu.VMEM((128,), jnp.float32),
        pltpu.VMEM((128,), jnp.float32),
        pltpu.VMEM((128,), jnp.float32),
        pltpu.VMEM((128,), jnp.float32),
        pltpu.VMEM((128,), jnp.float32),
        pltpu.VMEM((128,), jnp.float32),
        pltpu.VMEM((128,), jnp.float32),
        pltpu.VMEM((128,), jnp.float32),
        pltpu.VMEM((128,), jnp.float32),
        pltpu.VMEM((128,), jnp.float32),
        pltpu.VMEM((2, 2 * NPAD), jnp.int32),
        pltpu.VMEM((2, DET_PAD), jnp.float32),
        pltpu.SemaphoreType.DMA,
        pltpu.SemaphoreType.DMA,
        pltpu.SemaphoreType.DMA,
    ],
)
def _yolo_sc(outp_hbm, det_hbm, misc_hbm,
             buf, x1u, y1u, x2u, y2u, cfa, cpa, sma,
             x1o, y1o, x2o, y2o, ara, misc, db,
             sem_in, sem_d, sem_m):
    wid = lax.axis_index("s") * 2 + lax.axis_index("c")
    img_a = wid * 2
    lane = jnp.arange(16, dtype=jnp.int32)

    h_in = pltpu.async_copy(outp_hbm.at[pl.ds(img_a, 2)], buf, sem_in)
    h_in.wait()

    def decode_body(i, _):
        k = i // 4
        cb = (i % 4) * 16
        g = lane + cb

        def ld(f):
            return buf[k, pl.ds(f * NPAD + cb, 16)]

        conf0 = ld(4)
        conf1 = ld(9)
        use1 = conf1 > conf0
        conf = jnp.maximum(conf0, conf1)
        bx = jnp.where(use1, ld(5), ld(0))
        by = jnp.where(use1, ld(6), ld(1))
        bw = jnp.where(use1, ld(7), ld(2))
        bh = jnp.where(use1, ld(8), ld(3))
        colf = (g % S).astype(jnp.float32)
        rowf = (g // S).astype(jnp.float32)
        cx = (bx + colf) * GRID
        cy = (by + rowf) * GRID
        w = bw * WIMG
        h = bh * WIMG
        x1 = cx - w * 0.5
        y1 = cy - h * 0.5
        x2 = cx + w * 0.5
        y2 = cy + h * 0.5
        best = ld(10)
        bidx = jnp.zeros((16,), jnp.int32)
        for kk in range(1, 20):
            v = ld(10 + kk)
            bidx = jnp.where(v > best, kk, bidx)
            best = jnp.maximum(best, v)
        valid = (conf > CONF_THRES) & (g < NCELL)
        offv = bidx.astype(jnp.float32) * (2.0 * WIMG + 1.0)
        xo1 = x1 + offv
        xo2 = x2 + offv
        yo1 = y1 + offv
        yo2 = y2 + offv
        area = jnp.maximum(xo2 - xo1, 0.0) * jnp.maximum(yo2 - yo1, 0.0)
        sl = pl.ds(i * 16, 16)
        x1u[sl] = x1
        y1u[sl] = y1
        x2u[sl] = x2
        y2u[sl] = y2
        cfa[sl] = conf
        cpa[sl] = best
        sma[sl] = jnp.where(valid, conf, NEG_INF)
        x1o[sl] = xo1
        y1o[sl] = yo1
        x2o[sl] = xo2
        y2o[sl] = yo2
        ara[sl] = area
        misc[k, pl.ds(cb, 16)] = bidx
        misc[k, pl.ds(NPAD + cb, 16)] = valid.astype(jnp.int32)
        return 0

    lax.fori_loop(0, 8, decode_body, 0)

    sm0 = tuple(sma[pl.ds(i * 16, 16)] for i in range(8))

    def nms_body(_, carry):
        out_sm = []
        for k in range(2):
            koff = k * 64
            sm = carry[k * 4:k * 4 + 4]
            s0, s1, s2, s3 = sm
            mx = jnp.max(jnp.maximum(jnp.maximum(s0, s1), jnp.maximum(s2, s3)))
            cands = [
                jnp.where(s_c == mx, lane + c * 16, 999)
                for c, s_c in enumerate(sm)
            ]
            jstar = jnp.min(jnp.minimum(jnp.minimum(cands[0], cands[1]),
                                        jnp.minimum(cands[2], cands[3])))
            jv = jnp.full((16,), jstar + koff, jnp.int32)
            x1c = plsc.load_gather(x1o, [jv])
            y1c = plsc.load_gather(y1o, [jv])
            x2c = plsc.load_gather(x2o, [jv])
            y2c = plsc.load_gather(y2o, [jv])
            arc = plsc.load_gather(ara, [jv])
            kcur = plsc.load_gather(misc, [jnp.full((16,), k, jnp.int32),
                                           jnp.full((16,), jstar + NPAD,
                                                    jnp.int32)]) != 0
            for c, s_c in enumerate(sm):
                idxs = lane + c * 16
                unproc = (s_c != NEG_INF) & (idxs != jstar)
                sl = pl.ds(koff + c * 16, 16)
                ksl = pl.ds(NPAD + c * 16, 16)
                xx1 = jnp.maximum(x1o[sl], x1c)
                yy1 = jnp.maximum(y1o[sl], y1c)
                xx2 = jnp.minimum(x2o[sl], x2c)
                yy2 = jnp.minimum(y2o[sl], y2c)
                inter = (jnp.maximum(xx2 - xx1, 0.0)
                         * jnp.maximum(yy2 - yy1, 0.0))
                union = ara[sl] + arc - inter
                iou = inter / jnp.maximum(union, 1e-9)
                sup = (iou > NMS_THRES) & unproc & kcur
                misc[k, ksl] = jnp.where(sup, 0, misc[k, ksl])
                out_sm.append(jnp.where(idxs == jstar, NEG_INF, s_c))
        return tuple(out_sm)

    lax.fori_loop(0, NCELL, nms_body, sm0)

    def fin_body(i, _):
        k = i // 4
        cb = (i % 4) * 16
        g = lane + cb
        gc = jnp.minimum(g, NCELL - 1)
        m49 = g < NCELL
        sl = pl.ds(i * 16, 16)
        kv = misc[k, pl.ds(NPAD + cb, 16)] != 0
        kvec = jnp.full((16,), k, jnp.int32)
        for f, arr in enumerate((x1u, y1u, x2u, y2u, cfa, cpa)):
            plsc.store_scatter(db, [kvec, gc * 6 + f],
                               jnp.where(kv, arr[sl], 0.0), mask=m49)
        return 0

    lax.fori_loop(0, 8, fin_body, 0)
    pltpu.async_copy(db, det_hbm.at[pl.ds(img_a, 2)], sem_d).wait()
    pltpu.async_copy(misc, misc_hbm.at[pl.ds(img_a, 2)], sem_m).wait()


def kernel(images, outputs, prefix=0):
    del images, prefix
    outp_t = jnp.pad(outputs.reshape(BATCH, NCELL, D),
                     ((0, 0), (0, NPAD - NCELL), (0, 0)))
    outp_t = outp_t.transpose(0, 2, 1).reshape(BATCH, D * NPAD)
    det_p, misc_p = _yolo_sc(outp_t)
    det = det_p[:, : NCELL * 6].reshape(BATCH, NCELL, 6)
    return det, misc_p[:, :NCELL], misc_p[:, NPAD:NPAD + NCELL] != 0
